# baseline (speedup 1.0000x reference)
import functools

import jax
import jax.numpy as jnp
from jax import lax
from jax.experimental import pallas as pl
from jax.experimental.pallas import tpu as pltpu
from jax.experimental.pallas import tpu_sc as plsc

_TILE = 1024
_NBUF = 4
_SC_CHUNK = 256
_LANES = 16


def _logits_body(temp_ref, feat_hbm, text_hbm, fmu_ref, tmu_ref, cmu_ref,
                 logits_ref, feat_buf, text_buf, feat_sem, text_sem):
    i = pl.program_id(0)
    nsteps = pl.num_programs(0)

    def copy_in(tile, slot):
        pltpu.make_async_copy(
            feat_hbm.at[pl.ds(tile * _TILE, _TILE), :],
            feat_buf.at[slot], feat_sem.at[slot]).start()
        pltpu.make_async_copy(
            text_hbm.at[pl.ds(tile * _TILE, _TILE), :],
            text_buf.at[slot], text_sem.at[slot]).start()

    @pl.when(i == 0)
    def _():
        for k in range(_NBUF - 1):
            copy_in(k, k)

    nxt = i + _NBUF - 1

    @pl.when(nxt < nsteps)
    def _():
        copy_in(nxt, jax.lax.rem(nxt, _NBUF))

    slot = jax.lax.rem(i, _NBUF)
    pltpu.make_async_copy(
        feat_hbm.at[pl.ds(i * _TILE, _TILE), :],
        feat_buf.at[slot], feat_sem.at[slot]).wait()
    pltpu.make_async_copy(
        text_hbm.at[pl.ds(i * _TILE, _TILE), :],
        text_buf.at[slot], text_sem.at[slot]).wait()

    p1 = jnp.dot(feat_buf[slot], fmu_ref[...],
                 preferred_element_type=jnp.float32)
    p2 = jnp.dot(text_buf[slot], tmu_ref[...],
                 preferred_element_type=jnp.float32)
    combined = jnp.concatenate([p1, p2], axis=1)
    logits = jnp.dot(combined, cmu_ref[...],
                     preferred_element_type=jnp.float32)

    eff_temp = jnp.clip(temp_ref[0], 0.5, 5.0)
    logits_ref[...] = jnp.clip(logits / eff_temp, -20.0, 20.0)


def _tc_logits(feature, text_embedding, feature_mu, text_mu, combined_mu,
               temperature):
    tokens, dmodel = feature.shape
    nproj = feature_mu.shape[1]
    nexp = combined_mu.shape[1]

    return pl.pallas_call(
        _logits_body,
        grid=(tokens // _TILE,),
        in_specs=[
            pl.BlockSpec(memory_space=pltpu.SMEM),
            pl.BlockSpec(memory_space=pl.ANY),
            pl.BlockSpec(memory_space=pl.ANY),
            pl.BlockSpec((dmodel, nproj), lambda i: (0, 0)),
            pl.BlockSpec((dmodel, nproj), lambda i: (0, 0)),
            pl.BlockSpec((2 * nproj, nexp), lambda i: (0, 0)),
        ],
        out_specs=pl.BlockSpec((_TILE, nexp), lambda i: (i, 0)),
        out_shape=jax.ShapeDtypeStruct((tokens, nexp), jnp.float32),
        scratch_shapes=[
            pltpu.VMEM((_NBUF, _TILE, dmodel), jnp.float32),
            pltpu.VMEM((_NBUF, _TILE, dmodel), jnp.float32),
            pltpu.SemaphoreType.DMA((_NBUF,)),
            pltpu.SemaphoreType.DMA((_NBUF,)),
        ],
        compiler_params=pltpu.CompilerParams(
            dimension_semantics=("arbitrary",),
        ),
    )(temperature, feature, text_embedding, feature_mu, text_mu, combined_mu)


def _sc_onehot(logits):
    tokens, nexp = logits.shape
    info = plsc.get_sparse_core_info()
    nc, ns = info.num_cores, info.num_subcores
    nw = nc * ns
    rows_w = tokens // nw
    mesh = plsc.VectorSubcoreMesh(core_axis_name="c", subcore_axis_name="s")

    @functools.partial(
        pl.kernel,
        out_type=jax.ShapeDtypeStruct((tokens, nexp), jnp.float32),
        mesh=mesh,
        scratch_types=[
            pltpu.VMEM((_SC_CHUNK, nexp), jnp.float32),
            pltpu.VMEM((_SC_CHUNK, nexp), jnp.float32),
        ],
        compiler_params=pltpu.CompilerParams(needs_layout_passes=False),
    )
    def sc_kernel(logits_hbm, onehot_hbm, in_buf, out_buf):
        wid = lax.axis_index("s") * nc + lax.axis_index("c")
        base = wid * rows_w
        lane = lax.iota(jnp.int32, _LANES)

        def group(g, carry):
            rows = g * _LANES + lane
            m = plsc.load_gather(in_buf,
                                 [rows, jnp.zeros((_LANES,), jnp.int32)])
            best = jnp.zeros((_LANES,), jnp.int32)
            for c in range(1, nexp):
                v = plsc.load_gather(in_buf,
                                     [rows, jnp.full((_LANES,), c, jnp.int32)])
                upd = v > m
                m = jnp.where(upd, v, m)
                best = jnp.where(upd, jnp.full((_LANES,), c, jnp.int32), best)
            for c in range(nexp):
                oh = jnp.where(best == c, jnp.float32(1.0), jnp.float32(0.0))
                plsc.store_scatter(
                    out_buf, [rows, jnp.full((_LANES,), c, jnp.int32)], oh)
            return carry

        for ch in range(rows_w // _SC_CHUNK):
            r0 = base + ch * _SC_CHUNK
            pltpu.sync_copy(logits_hbm.at[pl.ds(r0, _SC_CHUNK), :], in_buf)
            lax.fori_loop(0, _SC_CHUNK // _LANES, group, 0)
            pltpu.sync_copy(out_buf, onehot_hbm.at[pl.ds(r0, _SC_CHUNK), :])

    return sc_kernel(logits)


def kernel(feature, text_embedding, feature_mu, text_mu, combined_mu,
           temperature):
    logits = _tc_logits(feature, text_embedding, feature_mu, text_mu,
                        combined_mu, temperature)
    onehot = _sc_onehot(logits)
    return (onehot, logits)


# speedup vs baseline: 1.7662x; 1.7662x over previous
import jax
import jax.numpy as jnp
from jax.experimental import pallas as pl
from jax.experimental.pallas import tpu as pltpu

_TILE = 1024
_NBUF = 4


def _router_body(temp_ref, feat_hbm, text_hbm, fmu_ref, tmu_ref, cmu_ref,
                 onehot_ref, logits_ref, feat_buf, text_buf, feat_sem,
                 text_sem):
    i = pl.program_id(0)
    nsteps = pl.num_programs(0)

    def copy_in(tile, slot):
        pltpu.make_async_copy(
            feat_hbm.at[pl.ds(tile * _TILE, _TILE), :],
            feat_buf.at[slot], feat_sem.at[slot]).start()
        pltpu.make_async_copy(
            text_hbm.at[pl.ds(tile * _TILE, _TILE), :],
            text_buf.at[slot], text_sem.at[slot]).start()

    @pl.when(i == 0)
    def _():
        for k in range(_NBUF - 1):
            copy_in(k, k)

    nxt = i + _NBUF - 1

    @pl.when(nxt < nsteps)
    def _():
        copy_in(nxt, jax.lax.rem(nxt, _NBUF))

    slot = jax.lax.rem(i, _NBUF)
    pltpu.make_async_copy(
        feat_hbm.at[pl.ds(i * _TILE, _TILE), :],
        feat_buf.at[slot], feat_sem.at[slot]).wait()
    pltpu.make_async_copy(
        text_hbm.at[pl.ds(i * _TILE, _TILE), :],
        text_buf.at[slot], text_sem.at[slot]).wait()

    p1 = jnp.dot(feat_buf[slot], fmu_ref[...],
                 preferred_element_type=jnp.float32)
    p2 = jnp.dot(text_buf[slot], tmu_ref[...],
                 preferred_element_type=jnp.float32)
    combined = jnp.concatenate([p1, p2], axis=1)
    logits = jnp.dot(combined, cmu_ref[...],
                     preferred_element_type=jnp.float32)

    eff_temp = jnp.clip(temp_ref[0], 0.5, 5.0)
    logits = jnp.clip(logits / eff_temp, -20.0, 20.0)
    logits_ref[...] = logits

    n = logits.shape[1]
    col = jax.lax.broadcasted_iota(jnp.int32, logits.shape, 1)
    row_max = jnp.max(logits, axis=1, keepdims=True)
    first_arg = jnp.min(jnp.where(logits == row_max, col, n), axis=1,
                        keepdims=True)
    onehot_ref[...] = (col == first_arg).astype(jnp.float32)


def kernel(feature, text_embedding, feature_mu, text_mu, combined_mu,
           temperature):
    tokens, dmodel = feature.shape
    nproj = feature_mu.shape[1]
    nexp = combined_mu.shape[1]
    grid = (tokens // _TILE,)

    onehot, logits = pl.pallas_call(
        _router_body,
        grid=grid,
        in_specs=[
            pl.BlockSpec(memory_space=pltpu.SMEM),
            pl.BlockSpec(memory_space=pl.ANY),
            pl.BlockSpec(memory_space=pl.ANY),
            pl.BlockSpec((dmodel, nproj), lambda i: (0, 0)),
            pl.BlockSpec((dmodel, nproj), lambda i: (0, 0)),
            pl.BlockSpec((2 * nproj, nexp), lambda i: (0, 0)),
        ],
        out_specs=[
            pl.BlockSpec((_TILE, nexp), lambda i: (i, 0)),
            pl.BlockSpec((_TILE, nexp), lambda i: (i, 0)),
        ],
        out_shape=[
            jax.ShapeDtypeStruct((tokens, nexp), jnp.float32),
            jax.ShapeDtypeStruct((tokens, nexp), jnp.float32),
        ],
        scratch_shapes=[
            pltpu.VMEM((_NBUF, _TILE, dmodel), jnp.float32),
            pltpu.VMEM((_NBUF, _TILE, dmodel), jnp.float32),
            pltpu.SemaphoreType.DMA((_NBUF,)),
            pltpu.SemaphoreType.DMA((_NBUF,)),
        ],
        compiler_params=pltpu.CompilerParams(
            dimension_semantics=("arbitrary",),
        ),
    )(temperature, feature, text_embedding, feature_mu, text_mu, combined_mu)
    return (onehot, logits)
